# baseline (speedup 1.0000x reference)
import functools

import jax
import jax.numpy as jnp
from jax import lax
from jax.experimental import pallas as pl
from jax.experimental.pallas import tpu as pltpu
import jax.experimental.pallas.tpu_sc as plsc

_NC = 2
_NS = 16
_CHUNK = 128
_ZROWS = 640
_GRP = 16


def _sc_segsum(n0, n1, d):
    acc_rows = _NS * _ZROWS
    nmax = max(n0, n1)
    mesh = plsc.VectorSubcoreMesh(core_axis_name="c", subcore_axis_name="s")
    scratch = [
        pltpu.VMEM((nmax, _CHUNK), jnp.int32),
        pltpu.VMEM((nmax, _CHUNK), jnp.int32),
        pltpu.VMEM((_CHUNK, d), jnp.float32),
        pltpu.VMEM_SHARED((acc_rows, d), jnp.float32),
        pltpu.SemaphoreType.DMA,
    ]

    @functools.partial(
        pl.kernel,
        out_type=jax.ShapeDtypeStruct((_NC, acc_rows, d), jnp.float32),
        mesh=mesh,
        scratch_types=scratch,
    )
    def k(h_hbm, s0_hbm, d0_hbm, s1_hbm, d1_hbm, zeros_hbm, out_hbm,
          sv, dv, val_v, acc, sem):
        c = lax.axis_index("c")
        s = lax.axis_index("s")
        pltpu.sync_copy(zeros_hbm, acc.at[pl.ds(s * _ZROWS, _ZROWS)])

        def run(src_hbm, dst_hbm, n_ch):
            pltpu.sync_copy(src_hbm.at[s], sv.at[pl.ds(0, n_ch)])
            pltpu.sync_copy(dst_hbm.at[s], dv.at[pl.ds(0, n_ch)])
            plsc.subcore_barrier()

            def step(j, carry):
                pltpu.async_copy(h_hbm.at[sv.at[j]], val_v, sem).wait()
                pltpu.sync_copy(val_v, acc.at[dv.at[j]], add=True)
                return carry

            lax.fori_loop(0, n_ch, step, 0)

        @pl.when(c == 0)
        def _():
            run(s0_hbm, d0_hbm, n0)

        @pl.when(c == 1)
        def _():
            run(s1_hbm, d1_hbm, n1)

        plsc.subcore_barrier()
        pltpu.sync_copy(acc.at[pl.ds(s * _ZROWS, _ZROWS)],
                        out_hbm.at[c, pl.ds(s * _ZROWS, _ZROWS)])

    return k


def _split_edges(adj, n, n0, n1):
    e = adj.shape[1]
    e_pad = _NS * (n0 + n1) * _CHUNK
    src = jnp.concatenate([adj[0], jnp.zeros((e_pad - e,), jnp.int32)])
    dst = jnp.concatenate([adj[1], jnp.full((e_pad - e,), n, jnp.int32)])
    cut = _NS * n0 * _CHUNK
    return (src[:cut].reshape(_NS, n0, _CHUNK),
            dst[:cut].reshape(_NS, n0, _CHUNK),
            src[cut:].reshape(_NS, n1, _CHUNK),
            dst[cut:].reshape(_NS, n1, _CHUNK))


def _sc_degree(n_ch):
    acc_rows = _NS * _ZROWS
    mesh = plsc.VectorSubcoreMesh(core_axis_name="c", subcore_axis_name="s")
    scratch = [
        pltpu.VMEM((n_ch, _CHUNK), jnp.int32),
        pltpu.VMEM((_CHUNK,), jnp.float32),
        pltpu.VMEM_SHARED((acc_rows,), jnp.float32),
        pltpu.SemaphoreType.DMA,
    ]

    @functools.partial(
        pl.kernel,
        out_type=jax.ShapeDtypeStruct((_NC * acc_rows,), jnp.float32),
        mesh=mesh,
        scratch_types=scratch,
    )
    def k(ones_hbm, dst_hbm, zeros_hbm, out_hbm, dst_v, val_v, acc, sem):
        c = lax.axis_index("c")
        s = lax.axis_index("s")
        pltpu.sync_copy(dst_hbm.at[c, s], dst_v)
        pltpu.sync_copy(ones_hbm, val_v)
        pltpu.sync_copy(zeros_hbm, acc.at[pl.ds(s * _ZROWS, _ZROWS)])
        plsc.subcore_barrier()

        def step(j, carry):
            pltpu.sync_copy(val_v, acc.at[dst_v.at[j]], add=True)
            return carry

        lax.fori_loop(0, n_ch, step, 0)
        plsc.subcore_barrier()
        pltpu.sync_copy(acc.at[pl.ds(s * _ZROWS, _ZROWS)],
                        out_hbm.at[pl.ds(c * acc_rows + s * _ZROWS, _ZROWS)])

    return k


def _tc_layer1(x, w1, b1, degp):
    n, f = x.shape
    h = w1.shape[1]
    rb = 1000

    def body(x_ref, w_ref, b_ref, deg_ref, h1_ref, isd_ref):
        deg = deg_ref[0] + deg_ref[1]
        isd = lax.rsqrt(jnp.maximum(deg, 1.0))
        acts = jnp.dot(x_ref[...], w_ref[...],
                       preferred_element_type=jnp.float32) + b_ref[...]
        h1_ref[...] = acts * isd
        isd_ref[...] = isd

    return pl.pallas_call(
        body,
        grid=(n // rb,),
        in_specs=[
            pl.BlockSpec((rb, f), lambda i: (i, 0)),
            pl.BlockSpec((f, h), lambda i: (0, 0)),
            pl.BlockSpec((1, h), lambda i: (0, 0)),
            pl.BlockSpec((_NC, rb, 1), lambda i: (0, i, 0)),
        ],
        out_specs=[
            pl.BlockSpec((rb, h), lambda i: (i, 0)),
            pl.BlockSpec((rb, 1), lambda i: (i, 0)),
        ],
        out_shape=[
            jax.ShapeDtypeStruct((n, h), jnp.float32),
            jax.ShapeDtypeStruct((n, 1), jnp.float32),
        ],
    )(x, w1, b1.reshape(1, -1), degp)


def _tc_layer2(aggp, isd, w2, b2):
    n = isd.shape[0]
    h = w2.shape[0]
    c = w2.shape[1]
    rb = 1000

    def body(agg_ref, isd_ref, w_ref, b_ref, out_ref):
        m = jnp.maximum((agg_ref[0] + agg_ref[1]) * isd_ref[...], 0.0)
        acts = jnp.dot(m, w_ref[...],
                       preferred_element_type=jnp.float32) + b_ref[...]
        out_ref[...] = acts * isd_ref[...]

    return pl.pallas_call(
        body,
        grid=(n // rb,),
        in_specs=[
            pl.BlockSpec((_NC, rb, h), lambda i: (0, i, 0)),
            pl.BlockSpec((rb, 1), lambda i: (i, 0)),
            pl.BlockSpec((h, c), lambda i: (0, 0)),
            pl.BlockSpec((1, c), lambda i: (0, 0)),
        ],
        out_specs=pl.BlockSpec((rb, c), lambda i: (i, 0)),
        out_shape=jax.ShapeDtypeStruct((n, c), jnp.float32),
    )(aggp, isd, w2, b2.reshape(1, -1))


def _tc_logsoftmax(aggp, isd, c):
    n = isd.shape[0]
    rb = 1000

    cp = aggp.shape[-1]

    def body(agg_ref, isd_ref, out_ref):
        v = ((agg_ref[0] + agg_ref[1]) * isd_ref[...])[:, :c]
        m = jnp.max(v, axis=1, keepdims=True)
        ex = jnp.exp(v - m)
        lse = jnp.log(jnp.sum(ex, axis=1, keepdims=True))
        out_ref[...] = v - m - lse

    return pl.pallas_call(
        body,
        grid=(n // rb,),
        in_specs=[
            pl.BlockSpec((_NC, rb, cp), lambda i: (0, i, 0)),
            pl.BlockSpec((rb, 1), lambda i: (i, 0)),
        ],
        out_specs=pl.BlockSpec((rb, c), lambda i: (i, 0)),
        out_shape=jax.ShapeDtypeStruct((n, c), jnp.float32),
    )(aggp, isd)


def kernel(x, adj, W1, b1, W2, b2):
    n, f = x.shape
    h = W1.shape[1]
    c = W2.shape[1]
    e = adj.shape[1]

    n_pair = -(-e // (_NS * _CHUNK))
    n0 = max(1, min(n_pair - 1, round(n_pair * 0.55)))
    n1 = n_pair - n0
    s0, d0, s1, d1 = _split_edges(adj, n, n0, n1)

    per_tile = -(-e // (_NC * _NS))
    n_ch = -(-per_tile // _CHUNK)
    e_pad = _NC * _NS * n_ch * _CHUNK
    dstd = jnp.concatenate(
        [adj[1], jnp.full((e_pad - e,), n, jnp.int32)]).reshape(
            _NC, _NS, n_ch, _CHUNK)

    cp = 128
    w2p = jnp.pad(W2, ((0, 0), (0, cp - c)))
    b2p = jnp.pad(b2, ((0, cp - c),))

    ones1 = jnp.ones((_CHUNK,), jnp.float32)
    zeros1 = jnp.zeros((_ZROWS,), jnp.float32)
    zeros_h = jnp.zeros((_ZROWS, h), jnp.float32)
    zeros_c = jnp.zeros((_ZROWS, cp), jnp.float32)

    acc_rows = _NS * _ZROWS
    degp = _sc_degree(n_ch)(ones1, dstd, zeros1).reshape(_NC, acc_rows, 1)
    h1, isd = _tc_layer1(x, W1, b1, degp)
    agg1p = _sc_segsum(n0, n1, h)(h1, s0, d0, s1, d1, zeros_h)
    h2 = _tc_layer2(agg1p, isd, w2p, b2p)
    agg2p = _sc_segsum(n0, n1, cp)(h2, s0, d0, s1, d1, zeros_c)
    return _tc_logsoftmax(agg2p, isd, c)


# speedup vs baseline: 1.0100x; 1.0100x over previous
import functools

import jax
import jax.numpy as jnp
from jax import lax
from jax.experimental import pallas as pl
from jax.experimental.pallas import tpu as pltpu
import jax.experimental.pallas.tpu_sc as plsc

_NC = 2
_NS = 16
_CHUNK = 128
_ZROWS = 640


def _sc_segsum(n0, n1, d):
    acc_rows = _NS * _ZROWS
    nmax = max(n0, n1)
    mesh = plsc.VectorSubcoreMesh(core_axis_name="c", subcore_axis_name="s")
    scratch = [
        pltpu.VMEM((nmax, _CHUNK), jnp.int32),
        pltpu.VMEM((nmax, _CHUNK), jnp.int32),
        pltpu.VMEM((_CHUNK, d), jnp.float32),
        pltpu.VMEM_SHARED((acc_rows, d), jnp.float32),
        pltpu.SemaphoreType.DMA,
    ]

    @functools.partial(
        pl.kernel,
        out_type=jax.ShapeDtypeStruct((_NC, acc_rows, d), jnp.float32),
        mesh=mesh,
        scratch_types=scratch,
    )
    def k(h_hbm, s0_hbm, d0_hbm, s1_hbm, d1_hbm, zeros_hbm, out_hbm,
          sv, dv, val_v, acc, sem):
        c = lax.axis_index("c")
        s = lax.axis_index("s")
        pltpu.sync_copy(zeros_hbm, acc.at[pl.ds(s * _ZROWS, _ZROWS)])

        def run(src_hbm, dst_hbm, n_ch):
            pltpu.sync_copy(src_hbm.at[s], sv.at[pl.ds(0, n_ch)])
            pltpu.sync_copy(dst_hbm.at[s], dv.at[pl.ds(0, n_ch)])
            plsc.subcore_barrier()

            def step(j, carry):
                pltpu.async_copy(h_hbm.at[sv.at[j]], val_v, sem).wait()
                pltpu.sync_copy(val_v, acc.at[dv.at[j]], add=True)
                return carry

            lax.fori_loop(0, n_ch, step, 0)

        @pl.when(c == 0)
        def _():
            run(s0_hbm, d0_hbm, n0)

        @pl.when(c == 1)
        def _():
            run(s1_hbm, d1_hbm, n1)

        plsc.subcore_barrier()
        pltpu.sync_copy(acc.at[pl.ds(s * _ZROWS, _ZROWS)],
                        out_hbm.at[c, pl.ds(s * _ZROWS, _ZROWS)])

    return k


def _split_edges(adj, n, n0, n1):
    e = adj.shape[1]
    e_pad = _NS * (n0 + n1) * _CHUNK
    src = jnp.concatenate([adj[0], jnp.zeros((e_pad - e,), jnp.int32)])
    dst = jnp.concatenate([adj[1], jnp.full((e_pad - e,), n, jnp.int32)])
    cut = _NS * n0 * _CHUNK
    return (src[:cut].reshape(_NS, n0, _CHUNK),
            dst[:cut].reshape(_NS, n0, _CHUNK),
            src[cut:].reshape(_NS, n1, _CHUNK),
            dst[cut:].reshape(_NS, n1, _CHUNK))


def _sc_degree(n_ch):
    acc_rows = _NS * _ZROWS
    mesh = plsc.VectorSubcoreMesh(core_axis_name="c", subcore_axis_name="s")
    scratch = [
        pltpu.VMEM((n_ch, _CHUNK), jnp.int32),
        pltpu.VMEM((_CHUNK,), jnp.float32),
        pltpu.VMEM_SHARED((acc_rows,), jnp.float32),
        pltpu.SemaphoreType.DMA,
    ]

    @functools.partial(
        pl.kernel,
        out_type=jax.ShapeDtypeStruct((_NC * acc_rows,), jnp.float32),
        mesh=mesh,
        scratch_types=scratch,
    )
    def k(ones_hbm, dst_hbm, zeros_hbm, out_hbm, dst_v, val_v, acc, sem):
        c = lax.axis_index("c")
        s = lax.axis_index("s")
        pltpu.sync_copy(dst_hbm.at[c, s], dst_v)
        pltpu.sync_copy(ones_hbm, val_v)
        pltpu.sync_copy(zeros_hbm, acc.at[pl.ds(s * _ZROWS, _ZROWS)])
        plsc.subcore_barrier()

        def step(j, carry):
            pltpu.sync_copy(val_v, acc.at[dst_v.at[j]], add=True)
            return carry

        lax.fori_loop(0, n_ch, step, 0)
        plsc.subcore_barrier()
        pltpu.sync_copy(acc.at[pl.ds(s * _ZROWS, _ZROWS)],
                        out_hbm.at[pl.ds(c * acc_rows + s * _ZROWS, _ZROWS)])

    return k


def _tc_layer1(x, w1, b1, degp):
    n, f = x.shape
    h = w1.shape[1]
    rb = 1000

    def body(x_ref, w_ref, b_ref, deg_ref, h1_ref, isd_ref):
        deg = deg_ref[0] + deg_ref[1]
        isd = lax.rsqrt(jnp.maximum(deg, 1.0))
        acts = jnp.dot(x_ref[...], w_ref[...],
                       preferred_element_type=jnp.float32) + b_ref[...]
        h1_ref[...] = acts * isd
        isd_ref[...] = isd

    return pl.pallas_call(
        body,
        grid=(n // rb,),
        in_specs=[
            pl.BlockSpec((rb, f), lambda i: (i, 0)),
            pl.BlockSpec((f, h), lambda i: (0, 0)),
            pl.BlockSpec((1, h), lambda i: (0, 0)),
            pl.BlockSpec((_NC, rb, 1), lambda i: (0, i, 0)),
        ],
        out_specs=[
            pl.BlockSpec((rb, h), lambda i: (i, 0)),
            pl.BlockSpec((rb, 1), lambda i: (i, 0)),
        ],
        out_shape=[
            jax.ShapeDtypeStruct((n, h), jnp.float32),
            jax.ShapeDtypeStruct((n, 1), jnp.float32),
        ],
    )(x, w1, b1.reshape(1, -1), degp)


def _tc_layer2(aggp, isd, w2, b2):
    n = isd.shape[0]
    h = w2.shape[0]
    c = w2.shape[1]
    rb = 1000

    def body(agg_ref, isd_ref, w_ref, b_ref, out_ref):
        m = jnp.maximum((agg_ref[0] + agg_ref[1]) * isd_ref[...], 0.0)
        acts = jnp.dot(m, w_ref[...],
                       preferred_element_type=jnp.float32) + b_ref[...]
        out_ref[...] = acts * isd_ref[...]

    return pl.pallas_call(
        body,
        grid=(n // rb,),
        in_specs=[
            pl.BlockSpec((_NC, rb, h), lambda i: (0, i, 0)),
            pl.BlockSpec((rb, 1), lambda i: (i, 0)),
            pl.BlockSpec((h, c), lambda i: (0, 0)),
            pl.BlockSpec((1, c), lambda i: (0, 0)),
        ],
        out_specs=pl.BlockSpec((rb, c), lambda i: (i, 0)),
        out_shape=jax.ShapeDtypeStruct((n, c), jnp.float32),
    )(aggp, isd, w2, b2.reshape(1, -1))


def _tc_logsoftmax(aggp, isd, c):
    n = isd.shape[0]
    rb = 1000

    cp = aggp.shape[-1]

    def body(agg_ref, isd_ref, out_ref):
        v = ((agg_ref[0] + agg_ref[1]) * isd_ref[...])[:, :c]
        m = jnp.max(v, axis=1, keepdims=True)
        ex = jnp.exp(v - m)
        lse = jnp.log(jnp.sum(ex, axis=1, keepdims=True))
        out_ref[...] = v - m - lse

    return pl.pallas_call(
        body,
        grid=(n // rb,),
        in_specs=[
            pl.BlockSpec((_NC, rb, cp), lambda i: (0, i, 0)),
            pl.BlockSpec((rb, 1), lambda i: (i, 0)),
        ],
        out_specs=pl.BlockSpec((rb, c), lambda i: (i, 0)),
        out_shape=jax.ShapeDtypeStruct((n, c), jnp.float32),
    )(aggp, isd)


def kernel(x, adj, W1, b1, W2, b2):
    n, f = x.shape
    h = W1.shape[1]
    c = W2.shape[1]
    e = adj.shape[1]

    n_pair = -(-e // (_NS * _CHUNK))
    n0 = max(1, min(n_pair - 1, round(n_pair * 0.60)))
    n1 = n_pair - n0
    s0, d0, s1, d1 = _split_edges(adj, n, n0, n1)

    per_tile = -(-e // (_NC * _NS))
    n_ch = -(-per_tile // _CHUNK)
    e_pad = _NC * _NS * n_ch * _CHUNK
    dstd = jnp.concatenate(
        [adj[1], jnp.full((e_pad - e,), n, jnp.int32)]).reshape(
            _NC, _NS, n_ch, _CHUNK)

    cp = 128
    w2p = jnp.pad(W2, ((0, 0), (0, cp - c)))
    b2p = jnp.pad(b2, ((0, cp - c),))

    ones1 = jnp.ones((_CHUNK,), jnp.float32)
    zeros1 = jnp.zeros((_ZROWS,), jnp.float32)
    zeros_h = jnp.zeros((_ZROWS, h), jnp.float32)
    zeros_c = jnp.zeros((_ZROWS, cp), jnp.float32)

    acc_rows = _NS * _ZROWS
    degp = _sc_degree(n_ch)(ones1, dstd, zeros1).reshape(_NC, acc_rows, 1)
    h1, isd = _tc_layer1(x, W1, b1, degp)
    agg1p = _sc_segsum(n0, n1, h)(h1, s0, d0, s1, d1, zeros_h)
    h2 = _tc_layer2(agg1p, isd, w2p, b2p)
    agg2p = _sc_segsum(n0, n1, cp)(h2, s0, d0, s1, d1, zeros_c)
    return _tc_logsoftmax(agg2p, isd, c)
